# Initial kernel scaffold; baseline (speedup 1.0000x reference)
import jax
import jax.numpy as jnp
from jax.experimental import pallas as pl


def kernel(x, edge_index, y, train_mask, W1, b1, Wl, bl, flow_mu, flow_log_sigma):
    raise NotImplementedError("write your pallas kernel here")


# speedup vs baseline: 17.9494x; 17.9494x over previous
import functools
import math

import jax
import jax.numpy as jnp
from jax import lax
from jax.experimental import pallas as pl
from jax.experimental.pallas import tpu as pltpu
from jax.experimental.pallas import tpu_sc as plsc

_N = 10000
_E = 320000
_D_IN = 128
_D_HID = 64
_D_LAT = 16
_C = 8
_CW = 16
_K_PROP = 10
_TELEPORT = 0.1

_NPAD = 10240
_NT = 16
_SL = _NPAD // _NT
_GE = 1024
_NG = 20
_EPADDED = _NT * _NG * _GE
_DUMMY = _NPAD - 1

_LOG2PI = math.log(2.0 * math.pi)
_LOG_SCALE = 0.5 * _D_LAT * math.log(4.0 * math.pi)


def _encode_body(x_ref, w1_ref, b1_ref, wl_ref, bl_ref, y_ref, m_ref,
                 mu_ref, iv_ref, slog_ref, out_ref):
    x = x_ref[...]
    w1 = w1_ref[...]
    ht = lax.dot_general(w1, x, (((0,), (1,)), ((), ())),
                         preferred_element_type=jnp.float32)
    ht = jnp.maximum(ht + b1_ref[...], 0.0)
    zt = lax.dot_general(wl_ref[...], ht, (((0,), (0,)), ((), ())),
                         preferred_element_type=jnp.float32)
    zt = zt + bl_ref[...]

    yv = y_ref[...]
    mf = m_ref[...]
    counts = [jnp.sum(jnp.where(yv == c, mf, 0.0)) for c in range(_C)]
    total = counts[0]
    for c in range(1, _C):
        total = total + counts[c]

    valid = lax.broadcasted_iota(jnp.int32, (1, _NPAD), 1) < _N
    for c in range(_C):
        acc = jnp.zeros((1, _NPAD), jnp.float32)
        for d in range(_D_LAT):
            m = mu_ref[d, c]
            iv = iv_ref[d, c]
            zrow = zt[d:d + 1, :]
            diff = zrow - m
            acc = acc + iv * (diff * diff)
        lq = -0.5 * acc - slog_ref[0, c] - 0.5 * _D_LAT * _LOG2PI
        pc = counts[c] / total
        lpc = jnp.log(jnp.full((1, _NPAD), pc, jnp.float32))
        lb = jnp.clip(lq + lpc + _LOG_SCALE, -30.0, 30.0)
        beta = jnp.exp(lb)
        out_ref[c:c + 1, :] = jnp.where(valid, beta, 0.0)
    out_ref[_C:_CW, :] = jnp.zeros((_CW - _C, _NPAD), jnp.float32)


def _prop_body(src_hbm, dst_hbm, h0_hbm, out_hbm,
               src_v, dst_v, msg_v, zeros_v, h0s_v, dexp_v,
               aggs_v, gtab, agg, gsem):
    wid = lax.axis_index("s")
    base = wid * _SL

    def _zfill(i, carry):
        zeros_v[i, :] = jnp.zeros((_CW,), jnp.float32)
        return carry
    lax.fori_loop(0, _SL, _zfill, 0)

    ones_v = msg_v.at[0]

    def _ofill(i, carry):
        ones_v[i, :] = jnp.ones((_CW,), jnp.float32)
        return carry
    lax.fori_loop(0, _GE, _ofill, 0)

    pltpu.sync_copy(src_hbm.at[wid], src_v)
    pltpu.sync_copy(dst_hbm.at[wid], dst_v)
    pltpu.sync_copy(h0_hbm.at[pl.ds(base, _SL)], h0s_v)

    pltpu.sync_copy(zeros_v, agg.at[pl.ds(base, _SL)])
    plsc.subcore_barrier()

    def _dchunk(j, carry):
        pltpu.sync_copy(ones_v, agg.at[dst_v.at[j]], add=True)
        return carry
    lax.fori_loop(0, _NG, _dchunk, 0)
    plsc.subcore_barrier()

    pltpu.sync_copy(agg.at[pl.ds(base, _SL)], aggs_v)

    def _dinvrow(i, carry):
        dg = aggs_v[i, :]
        xi = lax.bitcast_convert_type(dg, jnp.int32)
        yi = 0x5F3759DF - lax.shift_right_logical(xi, 1)
        y = lax.bitcast_convert_type(yi, jnp.float32)
        for _ in range(4):
            y = y * (1.5 - 0.5 * dg * y * y)
        y = jnp.where(dg > 0.5, y, 0.0)
        dexp_v[i, :] = y
        aggs_v[i, :] = y * h0s_v[i, :]
        return carry
    lax.fori_loop(0, _SL, _dinvrow, 0)
    pltpu.sync_copy(aggs_v, gtab.at[pl.ds(base, _SL)])

    for k in range(_K_PROP):
        pltpu.sync_copy(zeros_v, agg.at[pl.ds(base, _SL)])
        plsc.subcore_barrier()

        pltpu.async_copy(gtab.at[src_v.at[0]], msg_v.at[0], gsem)

        def _group(g, carry):
            b = lax.rem(g, 2)
            pltpu.make_async_copy(gtab.at[src_v.at[g]], msg_v.at[b],
                                  gsem).wait()

            @pl.when(g < _NG - 1)
            def _():
                pltpu.async_copy(gtab.at[src_v.at[g + 1]], msg_v.at[1 - b],
                                 gsem)

            pltpu.sync_copy(msg_v.at[b], agg.at[dst_v.at[g]], add=True)
            return carry
        lax.fori_loop(0, _NG, _group, 0)
        plsc.subcore_barrier()

        pltpu.sync_copy(agg.at[pl.ds(base, _SL)], aggs_v)

        def _upd(i, carry):
            a = aggs_v[i, :]
            d = dexp_v[i, :]
            hnew = (1.0 - _TELEPORT) * (d * a) + _TELEPORT * h0s_v[i, :]
            if k < _K_PROP - 1:
                aggs_v[i, :] = d * hnew
            else:
                aggs_v[i, :] = hnew
            return carry
        lax.fori_loop(0, _SL, _upd, 0)

        if k < _K_PROP - 1:
            pltpu.sync_copy(aggs_v, gtab.at[pl.ds(base, _SL)])
        else:
            pltpu.sync_copy(aggs_v, out_hbm.at[pl.ds(base, _SL)])


def _norm_body(h_ref, o_ref):
    alpha = h_ref[...] + 1.0
    s = jnp.sum(alpha, axis=1, keepdims=True)
    o_ref[...] = alpha / s


def kernel(x, edge_index, y, train_mask, W1, b1, Wl, bl, flow_mu,
           flow_log_sigma):
    f32 = jnp.float32
    xpad = jnp.pad(x, ((0, _NPAD - _N), (0, 0)))
    ypad = jnp.pad(y, (0, _NPAD - _N)).reshape(_NPAD // 128, 128)
    mpad = jnp.pad(train_mask.astype(f32), (0, _NPAD - _N)).reshape(
        _NPAD // 128, 128)
    b1r = b1.reshape(_D_HID, 1)
    blr = bl.reshape(_D_LAT, 1)
    muT = flow_mu.T
    ivT = jnp.exp(-2.0 * flow_log_sigma).T
    slog = jnp.sum(flow_log_sigma, axis=1).reshape(1, _C)

    betaT = pl.pallas_call(
        _encode_body,
        out_shape=jax.ShapeDtypeStruct((_CW, _NPAD), f32),
        in_specs=[
            pl.BlockSpec(memory_space=pltpu.VMEM),
            pl.BlockSpec(memory_space=pltpu.VMEM),
            pl.BlockSpec(memory_space=pltpu.VMEM),
            pl.BlockSpec(memory_space=pltpu.VMEM),
            pl.BlockSpec(memory_space=pltpu.VMEM),
            pl.BlockSpec(memory_space=pltpu.VMEM),
            pl.BlockSpec(memory_space=pltpu.VMEM),
            pl.BlockSpec(memory_space=pltpu.SMEM),
            pl.BlockSpec(memory_space=pltpu.SMEM),
            pl.BlockSpec(memory_space=pltpu.SMEM),
        ],
        out_specs=pl.BlockSpec(memory_space=pltpu.VMEM),
    )(xpad, W1, b1r, Wl, blr, ypad, mpad, muT, ivT, slog)

    h0p = betaT.T

    pad_e = _EPADDED - _E
    srcp = jnp.concatenate(
        [edge_index[0], jnp.full((pad_e,), _DUMMY, jnp.int32)]).reshape(
        _NT, _NG, _GE)
    dstp = jnp.concatenate(
        [edge_index[1], jnp.full((pad_e,), _DUMMY, jnp.int32)]).reshape(
        _NT, _NG, _GE)

    mesh = plsc.VectorSubcoreMesh(core_axis_name="c", subcore_axis_name="s",
                                  num_cores=1)
    hfin = pl.kernel(
        _prop_body,
        out_type=jax.ShapeDtypeStruct((_NPAD, _CW), f32),
        mesh=mesh,
        compiler_params=pltpu.CompilerParams(use_tc_tiling_on_sc=False),
        scratch_types=[
            pltpu.VMEM((_NG, _GE), jnp.int32),
            pltpu.VMEM((_NG, _GE), jnp.int32),
            pltpu.VMEM((2, _GE, _CW), f32),
            pltpu.VMEM((_SL, _CW), f32),
            pltpu.VMEM((_SL, _CW), f32),
            pltpu.VMEM((_SL, _CW), f32),
            pltpu.VMEM((_SL, _CW), f32),
            pltpu.HBM((_NPAD, _CW), f32),
            pltpu.VMEM_SHARED((_NPAD, _CW), f32),
            pltpu.SemaphoreType.DMA,
        ],
    )(srcp, dstp, h0p)

    soft = pl.pallas_call(
        _norm_body,
        out_shape=jax.ShapeDtypeStruct((_N, _C), f32),
        in_specs=[pl.BlockSpec(memory_space=pltpu.VMEM)],
        out_specs=pl.BlockSpec(memory_space=pltpu.VMEM),
    )(hfin[:_N, :_C])
    return soft


# speedup vs baseline: 34.2389x; 1.9075x over previous
import functools
import math

import jax
import jax.numpy as jnp
from jax import lax
from jax.experimental import pallas as pl
from jax.experimental.pallas import tpu as pltpu
from jax.experimental.pallas import tpu_sc as plsc

_N = 10000
_E = 320000
_D_IN = 128
_D_HID = 64
_D_LAT = 16
_C = 8
_CW = 16
_K_PROP = 10
_TELEPORT = 0.1

_NPAD = 10240
_NT = 16
_SL = _NPAD // _NT
_GE = 1024
_NG = 20
_EPADDED = _NT * _NG * _GE
_DUMMY = _NPAD - 1
_ZR = 128

_LOG2PI = math.log(2.0 * math.pi)
_LOG_SCALE = 0.5 * _D_LAT * math.log(4.0 * math.pi)


def _encode_body(x_ref, w1_ref, b1_ref, wl_ref, bl_ref, y_ref, m_ref,
                 mu_ref, iv_ref, slog_ref, out_ref):
    x = x_ref[...]
    w1 = w1_ref[...]
    ht = lax.dot_general(w1, x, (((0,), (1,)), ((), ())),
                         preferred_element_type=jnp.float32)
    ht = jnp.maximum(ht + b1_ref[...], 0.0)
    zt = lax.dot_general(wl_ref[...], ht, (((0,), (0,)), ((), ())),
                         preferred_element_type=jnp.float32)
    zt = zt + bl_ref[...]

    yv = y_ref[...]
    mf = m_ref[...]
    counts = [jnp.sum(jnp.where(yv == c, mf, 0.0)) for c in range(_C)]
    total = counts[0]
    for c in range(1, _C):
        total = total + counts[c]

    valid = lax.broadcasted_iota(jnp.int32, (1, _NPAD), 1) < _N
    for c in range(_C):
        acc = jnp.zeros((1, _NPAD), jnp.float32)
        for d in range(_D_LAT):
            m = mu_ref[d, c]
            iv = iv_ref[d, c]
            zrow = zt[d:d + 1, :]
            diff = zrow - m
            acc = acc + iv * (diff * diff)
        lq = -0.5 * acc - slog_ref[0, c] - 0.5 * _D_LAT * _LOG2PI
        pc = counts[c] / total
        lpc = jnp.log(jnp.full((1, _NPAD), pc, jnp.float32))
        lb = jnp.clip(lq + lpc + _LOG_SCALE, -30.0, 30.0)
        beta = jnp.exp(lb)
        out_ref[c:c + 1, :] = jnp.where(valid, beta, 0.0)
    out_ref[_C:_CW, :] = jnp.zeros((_CW - _C, _NPAD), jnp.float32)


def _prop_body(src_hbm, dst_hbm, h0_hbm, out_hbm,
               src_v, dst_v, msg0_v, msg1_v, zeros_v, h0s_v, dexp_v,
               aggs_v, gtab, agg, gsem):
    wid = lax.axis_index("s")
    base = wid * _SL

    def _zfill(i, carry):
        zeros_v[i, :] = jnp.zeros((_CW,), jnp.float32)
        return carry
    lax.fori_loop(0, _ZR, _zfill, 0)

    def _zero_agg_slice():
        def _zb(j, carry):
            pltpu.sync_copy(zeros_v, agg.at[pl.ds(base + j * _ZR, _ZR)])
            return carry
        lax.fori_loop(0, _SL // _ZR, _zb, 0)

    ones_v = msg0_v

    def _ofill(i, carry):
        ones_v[i, :] = jnp.ones((_CW,), jnp.float32)
        return carry
    lax.fori_loop(0, _GE, _ofill, 0)

    pltpu.sync_copy(src_hbm.at[wid], src_v)
    pltpu.sync_copy(dst_hbm.at[wid], dst_v)
    pltpu.sync_copy(h0_hbm.at[pl.ds(base, _SL)], h0s_v)

    _zero_agg_slice()
    plsc.subcore_barrier()

    def _dchunk(j, carry):
        pltpu.sync_copy(ones_v, agg.at[dst_v.at[j]], add=True)
        return carry
    lax.fori_loop(0, _NG, _dchunk, 0)
    plsc.subcore_barrier()

    pltpu.sync_copy(agg.at[pl.ds(base, _SL)], aggs_v)

    def _dinvrow(i, carry):
        dg = aggs_v[i, :]
        xi = lax.bitcast_convert_type(dg, jnp.int32)
        yi = 0x5F3759DF - lax.shift_right_logical(xi, 1)
        y = lax.bitcast_convert_type(yi, jnp.float32)
        for _ in range(4):
            y = y * (1.5 - 0.5 * dg * y * y)
        y = jnp.where(dg > 0.5, y, 0.0)
        dexp_v[i, :] = y
        aggs_v[i, :] = y * h0s_v[i, :]
        return carry
    lax.fori_loop(0, _SL, _dinvrow, 0)
    pltpu.sync_copy(aggs_v, gtab.at[pl.ds(base, _SL)])

    for k in range(_K_PROP):
        _zero_agg_slice()
        plsc.subcore_barrier()

        pltpu.async_copy(gtab.at[src_v.at[0]], msg0_v, gsem)

        def _pair(t, carry):
            g0 = 2 * t
            pltpu.make_async_copy(gtab.at[src_v.at[g0]], msg0_v, gsem).wait()
            pltpu.async_copy(gtab.at[src_v.at[g0 + 1]], msg1_v, gsem)
            pltpu.sync_copy(msg0_v, agg.at[dst_v.at[g0]], add=True)
            pltpu.make_async_copy(gtab.at[src_v.at[g0 + 1]], msg1_v,
                                  gsem).wait()

            @pl.when(t < _NG // 2 - 1)
            def _():
                pltpu.async_copy(gtab.at[src_v.at[g0 + 2]], msg0_v, gsem)

            pltpu.sync_copy(msg1_v, agg.at[dst_v.at[g0 + 1]], add=True)
            return carry
        lax.fori_loop(0, _NG // 2, _pair, 0)
        plsc.subcore_barrier()

        pltpu.sync_copy(agg.at[pl.ds(base, _SL)], aggs_v)

        def _upd(i, carry):
            a = aggs_v[i, :]
            d = dexp_v[i, :]
            hnew = (1.0 - _TELEPORT) * (d * a) + _TELEPORT * h0s_v[i, :]
            if k < _K_PROP - 1:
                aggs_v[i, :] = d * hnew
            else:
                aggs_v[i, :] = hnew
            return carry
        lax.fori_loop(0, _SL, _upd, 0)

        if k < _K_PROP - 1:
            pltpu.sync_copy(aggs_v, gtab.at[pl.ds(base, _SL)])
        else:
            pltpu.sync_copy(aggs_v, out_hbm.at[pl.ds(base, _SL)])


def _norm_body(h_ref, o_ref):
    alpha = h_ref[...] + 1.0
    s = jnp.sum(alpha, axis=1, keepdims=True)
    o_ref[...] = alpha / s


def kernel(x, edge_index, y, train_mask, W1, b1, Wl, bl, flow_mu,
           flow_log_sigma):
    f32 = jnp.float32
    xpad = jnp.pad(x, ((0, _NPAD - _N), (0, 0)))
    ypad = jnp.pad(y, (0, _NPAD - _N)).reshape(_NPAD // 128, 128)
    mpad = jnp.pad(train_mask.astype(f32), (0, _NPAD - _N)).reshape(
        _NPAD // 128, 128)
    b1r = b1.reshape(_D_HID, 1)
    blr = bl.reshape(_D_LAT, 1)
    muT = flow_mu.T
    ivT = jnp.exp(-2.0 * flow_log_sigma).T
    slog = jnp.sum(flow_log_sigma, axis=1).reshape(1, _C)

    betaT = pl.pallas_call(
        _encode_body,
        out_shape=jax.ShapeDtypeStruct((_CW, _NPAD), f32),
        in_specs=[
            pl.BlockSpec(memory_space=pltpu.VMEM),
            pl.BlockSpec(memory_space=pltpu.VMEM),
            pl.BlockSpec(memory_space=pltpu.VMEM),
            pl.BlockSpec(memory_space=pltpu.VMEM),
            pl.BlockSpec(memory_space=pltpu.VMEM),
            pl.BlockSpec(memory_space=pltpu.VMEM),
            pl.BlockSpec(memory_space=pltpu.VMEM),
            pl.BlockSpec(memory_space=pltpu.SMEM),
            pl.BlockSpec(memory_space=pltpu.SMEM),
            pl.BlockSpec(memory_space=pltpu.SMEM),
        ],
        out_specs=pl.BlockSpec(memory_space=pltpu.VMEM),
    )(xpad, W1, b1r, Wl, blr, ypad, mpad, muT, ivT, slog)

    h0p = betaT.T

    pad_e = _EPADDED - _E
    srcp = jnp.concatenate(
        [edge_index[0], jnp.full((pad_e,), _DUMMY, jnp.int32)]).reshape(
        _NT, _NG, _GE)
    dstp = jnp.concatenate(
        [edge_index[1], jnp.full((pad_e,), _DUMMY, jnp.int32)]).reshape(
        _NT, _NG, _GE)

    mesh = plsc.VectorSubcoreMesh(core_axis_name="c", subcore_axis_name="s",
                                  num_cores=1)
    hfin = pl.kernel(
        _prop_body,
        out_type=jax.ShapeDtypeStruct((_NPAD, _CW), f32),
        mesh=mesh,
        compiler_params=pltpu.CompilerParams(use_tc_tiling_on_sc=False),
        scratch_types=[
            pltpu.VMEM((_NG, _GE), jnp.int32),
            pltpu.VMEM((_NG, _GE), jnp.int32),
            pltpu.VMEM((_GE, _CW), f32),
            pltpu.VMEM((_GE, _CW), f32),
            pltpu.VMEM((_ZR, _CW), f32),
            pltpu.VMEM((_SL, _CW), f32),
            pltpu.VMEM((_SL, _CW), f32),
            pltpu.VMEM((_SL, _CW), f32),
            pltpu.VMEM_SHARED((_NPAD, _CW), f32),
            pltpu.VMEM_SHARED((_NPAD, _CW), f32),
            pltpu.SemaphoreType.DMA,
        ],
    )(srcp, dstp, h0p)

    soft = pl.pallas_call(
        _norm_body,
        out_shape=jax.ShapeDtypeStruct((_N, _C), f32),
        in_specs=[pl.BlockSpec(memory_space=pltpu.VMEM)],
        out_specs=pl.BlockSpec(memory_space=pltpu.VMEM),
    )(hfin[:_N, :_C])
    return soft


# speedup vs baseline: 47.4405x; 1.3856x over previous
import functools
import math

import jax
import jax.numpy as jnp
from jax import lax
from jax.experimental import pallas as pl
from jax.experimental.pallas import tpu as pltpu
from jax.experimental.pallas import tpu_sc as plsc

_N = 10000
_E = 320000
_D_IN = 128
_D_HID = 64
_D_LAT = 16
_C = 8
_CW = 8
_K_PROP = 10
_TELEPORT = 0.1

_NPAD = 10240
_NT = 16
_SL = _NPAD // _NT
_PR = _SL // 2
_GE = 1024
_NG = 20
_EPADDED = _NT * _NG * _GE
_DUMMY = _NPAD - 1
_ZR = 128

_LOG2PI = math.log(2.0 * math.pi)
_LOG_SCALE = 0.5 * _D_LAT * math.log(4.0 * math.pi)


def _encode_body(x_ref, w1_ref, b1_ref, wl_ref, bl_ref, y_ref, m_ref,
                 mu_ref, iv_ref, slog_ref, out_ref):
    x = x_ref[...]
    w1 = w1_ref[...]
    ht = lax.dot_general(w1, x, (((0,), (1,)), ((), ())),
                         preferred_element_type=jnp.float32)
    ht = jnp.maximum(ht + b1_ref[...], 0.0)
    zt = lax.dot_general(wl_ref[...], ht, (((0,), (0,)), ((), ())),
                         preferred_element_type=jnp.float32)
    zt = zt + bl_ref[...]

    yv = y_ref[...]
    mf = m_ref[...]
    counts = [jnp.sum(jnp.where(yv == c, mf, 0.0)) for c in range(_C)]
    total = counts[0]
    for c in range(1, _C):
        total = total + counts[c]

    valid = lax.broadcasted_iota(jnp.int32, (1, _NPAD), 1) < _N
    for c in range(_C):
        acc = jnp.zeros((1, _NPAD), jnp.float32)
        for d in range(_D_LAT):
            m = mu_ref[d, c]
            iv = iv_ref[d, c]
            zrow = zt[d:d + 1, :]
            diff = zrow - m
            acc = acc + iv * (diff * diff)
        lq = -0.5 * acc - slog_ref[0, c] - 0.5 * _D_LAT * _LOG2PI
        pc = counts[c] / total
        lpc = jnp.log(jnp.full((1, _NPAD), pc, jnp.float32))
        lb = jnp.clip(lq + lpc + _LOG_SCALE, -30.0, 30.0)
        beta = jnp.exp(lb)
        out_ref[c:c + 1, :] = jnp.where(valid, beta, 0.0)
    out_ref[_C:2 * _CW, :] = jnp.zeros((2 * _CW - _C, _NPAD), jnp.float32)


def _prop_body(src_hbm, dst_hbm, h0_hbm, zq_hbm, ones_hbm, out_hbm,
               src_v, dst_v, msg0_v, msg1_v, zeros_v, h0s_v, dexp_v,
               aggs_v, gnew_v, gtab, agg, gsem):
    wid = lax.axis_index("s")
    base = wid * _SL

    pltpu.sync_copy(zq_hbm, zeros_v)
    pltpu.sync_copy(ones_hbm, msg0_v)
    ones_v = msg0_v


    def _zero_agg_slice():
        def _zb(j, carry):
            pltpu.sync_copy(zeros_v, agg.at[pl.ds(base + j * _ZR, _ZR)])
            return carry
        lax.fori_loop(0, _SL // _ZR, _zb, 0)

    pltpu.sync_copy(src_hbm.at[wid], src_v)
    pltpu.sync_copy(dst_hbm.at[wid], dst_v)
    pltpu.sync_copy(h0_hbm.at[pl.ds(base, _SL)], h0s_v)

    _zero_agg_slice()
    plsc.subcore_barrier()

    def _dchunk(j, carry):
        pltpu.sync_copy(ones_v, agg.at[dst_v.at[j]], add=True)
        return carry
    lax.fori_loop(0, _NG, _dchunk, 0)
    plsc.subcore_barrier()

    pltpu.sync_copy(agg.at[pl.ds(base, _SL)],
                    aggs_v.at[:, pl.ds(0, _CW)])

    def _dinvrow(i, carry):
        dg = aggs_v[i, :]
        xi = lax.bitcast_convert_type(dg, jnp.int32)
        yi = 0x5F3759DF - lax.shift_right_logical(xi, 1)
        y = lax.bitcast_convert_type(yi, jnp.float32)
        for _ in range(4):
            y = y * (1.5 - 0.5 * dg * y * y)
        y = jnp.where(dg > 0.5, y, 0.0)
        dexp_v[i, :] = y
        gnew_v[i, :] = y * h0s_v[i, :]
        return carry
    lax.fori_loop(0, _SL, _dinvrow, 0)
    pltpu.sync_copy(gnew_v.at[:, pl.ds(0, _CW)],
                    gtab.at[pl.ds(base, _SL)])

    for k in range(_K_PROP):
        _zero_agg_slice()
        plsc.subcore_barrier()

        pltpu.async_copy(gtab.at[src_v.at[0]], msg0_v, gsem)

        def _pair(t, carry):
            g0 = 2 * t
            pltpu.make_async_copy(gtab.at[src_v.at[g0]], msg0_v, gsem).wait()
            pltpu.async_copy(gtab.at[src_v.at[g0 + 1]], msg1_v, gsem)
            pltpu.sync_copy(msg0_v, agg.at[dst_v.at[g0]], add=True)
            pltpu.make_async_copy(gtab.at[src_v.at[g0 + 1]], msg1_v,
                                  gsem).wait()

            @pl.when(t < _NG // 2 - 1)
            def _():
                pltpu.async_copy(gtab.at[src_v.at[g0 + 2]], msg0_v, gsem)

            pltpu.sync_copy(msg1_v, agg.at[dst_v.at[g0 + 1]], add=True)
            return carry
        lax.fori_loop(0, _NG // 2, _pair, 0)
        plsc.subcore_barrier()

        pltpu.sync_copy(agg.at[pl.ds(base, _SL)],
                        aggs_v.at[:, pl.ds(0, _CW)])

        def _upd(i, carry):
            a = aggs_v[i, :]
            d = dexp_v[i, :]
            hnew = (1.0 - _TELEPORT) * (d * a) + _TELEPORT * h0s_v[i, :]
            if k < _K_PROP - 1:
                gnew_v[i, :] = d * hnew
            else:
                gnew_v[i, :] = hnew
            return carry
        lax.fori_loop(0, _SL, _upd, 0)

        if k < _K_PROP - 1:
            pltpu.sync_copy(gnew_v.at[:, pl.ds(0, _CW)],
                            gtab.at[pl.ds(base, _SL)])
        else:
            pltpu.sync_copy(gnew_v, out_hbm.at[pl.ds(base, _SL)])


def _norm_body(h_ref, o_ref):
    alpha = h_ref[...] + 1.0
    s = jnp.sum(alpha, axis=1, keepdims=True)
    o_ref[...] = alpha / s


def kernel(x, edge_index, y, train_mask, W1, b1, Wl, bl, flow_mu,
           flow_log_sigma):
    f32 = jnp.float32
    xpad = jnp.pad(x, ((0, _NPAD - _N), (0, 0)))
    ypad = jnp.pad(y, (0, _NPAD - _N)).reshape(_NPAD // 128, 128)
    mpad = jnp.pad(train_mask.astype(f32), (0, _NPAD - _N)).reshape(
        _NPAD // 128, 128)
    b1r = b1.reshape(_D_HID, 1)
    blr = bl.reshape(_D_LAT, 1)
    muT = flow_mu.T
    ivT = jnp.exp(-2.0 * flow_log_sigma).T
    slog = jnp.sum(flow_log_sigma, axis=1).reshape(1, _C)

    betaT = pl.pallas_call(
        _encode_body,
        out_shape=jax.ShapeDtypeStruct((2 * _CW, _NPAD), f32),
        in_specs=[
            pl.BlockSpec(memory_space=pltpu.VMEM),
            pl.BlockSpec(memory_space=pltpu.VMEM),
            pl.BlockSpec(memory_space=pltpu.VMEM),
            pl.BlockSpec(memory_space=pltpu.VMEM),
            pl.BlockSpec(memory_space=pltpu.VMEM),
            pl.BlockSpec(memory_space=pltpu.VMEM),
            pl.BlockSpec(memory_space=pltpu.VMEM),
            pl.BlockSpec(memory_space=pltpu.SMEM),
            pl.BlockSpec(memory_space=pltpu.SMEM),
            pl.BlockSpec(memory_space=pltpu.SMEM),
        ],
        out_specs=pl.BlockSpec(memory_space=pltpu.VMEM),
    )(xpad, W1, b1r, Wl, blr, ypad, mpad, muT, ivT, slog)

    h0p = betaT.T

    pad_e = _EPADDED - _E
    srcp = jnp.concatenate(
        [edge_index[0], jnp.full((pad_e,), _DUMMY, jnp.int32)]).reshape(
        _NT, _NG, _GE)
    dstp = jnp.concatenate(
        [edge_index[1], jnp.full((pad_e,), _DUMMY, jnp.int32)]).reshape(
        _NT, _NG, _GE)

    mesh = plsc.VectorSubcoreMesh(core_axis_name="c", subcore_axis_name="s",
                                  num_cores=1)
    hfin = pl.kernel(
        _prop_body,
        out_type=jax.ShapeDtypeStruct((_NPAD, 2 * _CW), f32),
        mesh=mesh,
        compiler_params=pltpu.CompilerParams(use_tc_tiling_on_sc=False),
        scratch_types=[
            pltpu.VMEM((_NG, _GE), jnp.int32),
            pltpu.VMEM((_NG, _GE), jnp.int32),
            pltpu.VMEM((_GE, _CW), f32),
            pltpu.VMEM((_GE, _CW), f32),
            pltpu.VMEM((_ZR, _CW), f32),
            pltpu.VMEM((_SL, 2 * _CW), f32),
            pltpu.VMEM((_SL, 2 * _CW), f32),
            pltpu.VMEM((_SL, 2 * _CW), f32),
            pltpu.VMEM((_SL, 2 * _CW), f32),
            pltpu.VMEM_SHARED((_NPAD, _CW), f32),
            pltpu.VMEM_SHARED((_NPAD, _CW), f32),
            pltpu.SemaphoreType.DMA,
        ],
    )(srcp, dstp, h0p, jnp.zeros((_ZR, _CW), f32), jnp.ones((_GE, _CW), f32))

    soft = pl.pallas_call(
        _norm_body,
        out_shape=jax.ShapeDtypeStruct((_N, _C), f32),
        in_specs=[pl.BlockSpec(memory_space=pltpu.VMEM)],
        out_specs=pl.BlockSpec(memory_space=pltpu.VMEM),
    )(hfin[:_N, :_C])
    return soft
